# baseline (device time: 24430 ns/iter reference)
import jax
import jax.numpy as jnp
from jax import lax
from jax.experimental import pallas as pl
from jax.experimental.pallas import tpu as pltpu

N_DEV = 4
B, H, D, BS = 8, 8, 64, 16
PAGES_PER_SHARD = 64
T = PAGES_PER_SHARD * BS
ROWS = B * H
NEG_INF = -1e30


def kernel(Q, K, V, bt, lens):
    def body(q_ref, k_ref, v_ref, bt_ref, lens_ref, out_ref,
             comm_ref, send_sems, recv_sems):
        my_pos = lax.axis_index("i")
        left = (my_pos - 1) % N_DEV
        right = (my_pos + 1) % N_DEV

        qs = q_ref[...].reshape(B, H, D)
        kl = k_ref[...].reshape(T, H, D)
        vl = v_ref[...].reshape(T, H, D)
        bt_local = bt_ref[...]
        lens_col = lens_ref[...]

        base = my_pos * PAGES_PER_SHARD
        j_iota = lax.broadcasted_iota(jnp.int32, (B, 64, T), 1)
        k_iota = lax.broadcasted_iota(jnp.int32, (B, 64, T), 2)
        match = (bt_local[:, :, None] == (k_iota >> 4) + base) & (
            j_iota < lens_col[:, :, None]
        )
        w_tok = jnp.sum(match.astype(jnp.float32), axis=1)
        masked = w_tok > 0.0

        scale = D ** -0.5
        o_heads, m_heads, l_heads = [], [], []
        for h in range(H):
            qh = qs[:, h, :]
            kh = kl[:, h, :]
            vh = vl[:, h, :]
            s = lax.dot_general(
                qh, kh, (((1,), (1,)), ((), ())),
                preferred_element_type=jnp.float32,
            ) * scale
            s = jnp.where(masked, s, NEG_INF)
            m_h = jnp.max(s, axis=1, keepdims=True)
            p = jnp.exp(s - m_h) * w_tok
            l_h = jnp.sum(p, axis=1, keepdims=True)
            o_h = lax.dot_general(
                p, vh, (((1,), (0,)), ((), ())),
                preferred_element_type=jnp.float32,
            )
            o_heads.append(o_h)
            m_heads.append(m_h)
            l_heads.append(l_h)

        o_acc = jnp.stack(o_heads, axis=1).reshape(ROWS, D)
        m_bh = jnp.concatenate(m_heads, axis=1)
        l_bh = jnp.concatenate(l_heads, axis=1)
        m_acc = jnp.broadcast_to(m_bh[:, :, None], (B, H, D)).reshape(ROWS, D)
        l_acc = jnp.broadcast_to(l_bh[:, :, None], (B, H, D)).reshape(ROWS, D)
        comm_ref[0, :, :] = jnp.concatenate([o_acc, m_acc, l_acc], axis=1)

        barrier_sem = pltpu.get_barrier_semaphore()
        for nbr in [left, right]:
            pl.semaphore_signal(
                barrier_sem, inc=1,
                device_id=(nbr,), device_id_type=pl.DeviceIdType.MESH,
            )
        pl.semaphore_wait(barrier_sem, 2)

        for h in range(N_DEV - 1):
            send_slot = h % 2
            recv_slot = (h + 1) % 2
            rdma = pltpu.make_async_remote_copy(
                src_ref=comm_ref.at[send_slot],
                dst_ref=comm_ref.at[recv_slot],
                send_sem=send_sems.at[send_slot],
                recv_sem=recv_sems.at[recv_slot],
                device_id=(right,),
                device_id_type=pl.DeviceIdType.MESH,
            )
            rdma.start()
            rdma.wait()

            c = comm_ref[recv_slot]
            o_r = c[:, 0:D]
            m_r = c[:, D:2 * D]
            l_r = c[:, 2 * D:3 * D]
            m_new = jnp.maximum(m_acc, m_r)
            a = jnp.exp(m_acc - m_new)
            b = jnp.exp(m_r - m_new)
            o_acc = o_acc * a + o_r * b
            l_acc = l_acc * a + l_r * b
            m_acc = m_new

        out_ref[...] = (o_acc / l_acc).reshape(B, 1, H, D)

    return pl.pallas_call(
        body,
        out_shape=jax.ShapeDtypeStruct((B, 1, H, D), jnp.float32),
        in_specs=[
            pl.BlockSpec(memory_space=pltpu.VMEM),
            pl.BlockSpec(memory_space=pltpu.VMEM),
            pl.BlockSpec(memory_space=pltpu.VMEM),
            pl.BlockSpec(memory_space=pltpu.VMEM),
            pl.BlockSpec(memory_space=pltpu.VMEM),
        ],
        out_specs=pl.BlockSpec(memory_space=pltpu.VMEM),
        scratch_shapes=[
            pltpu.VMEM((2, ROWS, 3 * D), jnp.float32),
            pltpu.SemaphoreType.DMA((2,)),
            pltpu.SemaphoreType.DMA((2,)),
        ],
        compiler_params=pltpu.CompilerParams(collective_id=0),
    )(Q, K, V, bt, lens.reshape(B, 1))


# device time: 18845 ns/iter; 1.2964x vs baseline; 1.2964x over previous
import jax
import jax.numpy as jnp
from jax import lax
from jax.experimental import pallas as pl
from jax.experimental.pallas import tpu as pltpu

N_DEV = 4
B, H, D, BS = 8, 8, 64, 16
PAGES_PER_SHARD = 64
T = PAGES_PER_SHARD * BS
ROWS = B * H
NEG_INF = -1e30


def kernel(Q, K, V, bt, lens):
    def body(q_ref, k_ref, v_ref, bt_ref, lens_ref, out_ref,
             send_buf, recv_buf, send_sems, recv_sems):
        my_pos = lax.axis_index("i")

        qs = q_ref[...].reshape(B, H, D)
        kl = k_ref[...].reshape(T, H, D)
        vl = v_ref[...].reshape(T, H, D)
        bt_local = bt_ref[...]
        lens_col = lens_ref[...]

        base = my_pos * PAGES_PER_SHARD
        j_iota = lax.broadcasted_iota(jnp.int32, (B, 64, PAGES_PER_SHARD), 1)
        p_iota = lax.broadcasted_iota(jnp.int32, (B, 64, PAGES_PER_SHARD), 2)
        match = (bt_local[:, :, None] == p_iota + base) & (
            j_iota < lens_col[:, :, None]
        )
        w = jnp.sum(match.astype(jnp.float32), axis=1)
        w_tok = jnp.broadcast_to(
            w[:, :, None], (B, PAGES_PER_SHARD, BS)
        ).reshape(B, T)
        masked = w_tok > 0.0

        scale = D ** -0.5
        o_heads, m_heads, l_heads = [], [], []
        for h in range(H):
            qh = qs[:, h, :]
            kh = kl[:, h, :]
            vh = vl[:, h, :]
            s = lax.dot_general(
                qh, kh, (((1,), (1,)), ((), ())),
                preferred_element_type=jnp.float32,
            ) * scale
            s = jnp.where(masked, s, NEG_INF)
            m_h = jnp.max(s, axis=1, keepdims=True)
            p = jnp.exp(s - m_h) * w_tok
            l_h = jnp.sum(p, axis=1, keepdims=True)
            o_h = lax.dot_general(
                p, vh, (((1,), (0,)), ((), ())),
                preferred_element_type=jnp.float32,
            )
            o_heads.append(o_h)
            m_heads.append(m_h)
            l_heads.append(l_h)

        o_acc = jnp.stack(o_heads, axis=1).reshape(ROWS, D)
        m_bh = jnp.concatenate(m_heads, axis=1)
        l_bh = jnp.concatenate(l_heads, axis=1)
        m_acc = jnp.broadcast_to(m_bh[:, :, None], (B, H, D)).reshape(ROWS, D)
        l_acc = jnp.broadcast_to(l_bh[:, :, None], (B, H, D)).reshape(ROWS, D)
        send_buf[...] = jnp.concatenate([o_acc, m_acc, l_acc], axis=1)

        barrier_sem = pltpu.get_barrier_semaphore()
        for r in range(1, N_DEV):
            pl.semaphore_signal(
                barrier_sem, inc=1,
                device_id=((my_pos + r) % N_DEV,),
                device_id_type=pl.DeviceIdType.MESH,
            )
        pl.semaphore_wait(barrier_sem, N_DEV - 1)

        rdmas = []
        for r in range(1, N_DEV):
            rdma = pltpu.make_async_remote_copy(
                src_ref=send_buf,
                dst_ref=recv_buf.at[r - 1],
                send_sem=send_sems.at[r - 1],
                recv_sem=recv_sems.at[r - 1],
                device_id=((my_pos + r) % N_DEV,),
                device_id_type=pl.DeviceIdType.MESH,
            )
            rdma.start()
            rdmas.append(rdma)

        for j in (0, 2, 1):
            rdmas[j].wait_recv()
            c = recv_buf[j]
            o_r = c[:, 0:D]
            m_r = c[:, D:2 * D]
            l_r = c[:, 2 * D:3 * D]
            m_new = jnp.maximum(m_acc, m_r)
            a = jnp.exp(m_acc - m_new)
            b = jnp.exp(m_r - m_new)
            o_acc = o_acc * a + o_r * b
            l_acc = l_acc * a + l_r * b
            m_acc = m_new

        for j in range(N_DEV - 1):
            rdmas[j].wait_send()

        out_ref[...] = (o_acc / l_acc).reshape(B, 1, H, D)

    return pl.pallas_call(
        body,
        out_shape=jax.ShapeDtypeStruct((B, 1, H, D), jnp.float32),
        in_specs=[
            pl.BlockSpec(memory_space=pltpu.VMEM),
            pl.BlockSpec(memory_space=pltpu.VMEM),
            pl.BlockSpec(memory_space=pltpu.VMEM),
            pl.BlockSpec(memory_space=pltpu.VMEM),
            pl.BlockSpec(memory_space=pltpu.VMEM),
        ],
        out_specs=pl.BlockSpec(memory_space=pltpu.VMEM),
        scratch_shapes=[
            pltpu.VMEM((ROWS, 3 * D), jnp.float32),
            pltpu.VMEM((3, ROWS, 3 * D), jnp.float32),
            pltpu.SemaphoreType.DMA((3,)),
            pltpu.SemaphoreType.DMA((3,)),
        ],
        compiler_params=pltpu.CompilerParams(collective_id=0),
    )(Q, K, V, bt, lens.reshape(B, 1))


# device time: 15190 ns/iter; 1.6083x vs baseline; 1.2406x over previous
import jax
import jax.numpy as jnp
from jax import lax
from jax.experimental import pallas as pl
from jax.experimental.pallas import tpu as pltpu

N_DEV = 4
B, H, D, BS = 8, 8, 64, 16
PAGES_PER_SHARD = 64
T = PAGES_PER_SHARD * BS
HD = H * D
ROWS = B * H
NEG_INF = -1e30


def kernel(Q, K, V, bt, lens):
    def body(q_ref, k_ref, v_ref, bt_ref, lens_ref, out_ref,
             send_buf, recv_buf, send_sems, recv_sems):
        my_pos = lax.axis_index("i")

        q2 = q_ref[...]
        k2 = k_ref[...]
        v2 = v_ref[...]
        bt_local = bt_ref[...]
        lens_col = lens_ref[...]

        base = my_pos * PAGES_PER_SHARD
        j_iota = lax.broadcasted_iota(jnp.int32, (B, 64, PAGES_PER_SHARD), 1)
        p_iota = lax.broadcasted_iota(jnp.int32, (B, 64, PAGES_PER_SHARD), 2)
        match = (bt_local[:, :, None] == p_iota + base) & (
            j_iota < lens_col[:, :, None]
        )
        w = jnp.sum(match.astype(jnp.float32), axis=1)
        w_tok = jnp.broadcast_to(
            w[:, :, None], (B, PAGES_PER_SHARD, BS)
        ).reshape(B, T)
        w_tok_r = jnp.broadcast_to(
            w_tok[:, None, :], (B, H, T)
        ).reshape(ROWS, T)
        masked_r = w_tok_r > 0.0

        qb = jnp.broadcast_to(q2[:, None, :], (B, H, HD)).reshape(ROWS, HD)
        row_h = lax.broadcasted_iota(jnp.int32, (ROWS, HD), 0) % H
        lane_h = lax.broadcasted_iota(jnp.int32, (ROWS, HD), 1) >> 6
        head_mask = row_h == lane_h
        q2m = jnp.where(head_mask, qb, 0.0)

        s = lax.dot_general(
            q2m, k2, (((1,), (1,)), ((), ())),
            preferred_element_type=jnp.float32,
        ) * (D ** -0.5)
        s = jnp.where(masked_r, s, NEG_INF)
        m_col = jnp.max(s, axis=1, keepdims=True)
        p = jnp.exp(s - m_col) * w_tok_r
        l_col = jnp.sum(p, axis=1, keepdims=True)
        o_full = lax.dot_general(
            p, v2, (((1,), (0,)), ((), ())),
            preferred_element_type=jnp.float32,
        )

        row_h64 = lax.broadcasted_iota(jnp.int32, (ROWS, D), 0) % H
        o_acc = jnp.zeros((ROWS, D), jnp.float32)
        for h in range(H):
            o_acc = o_acc + jnp.where(
                row_h64 == h, o_full[:, h * D:(h + 1) * D], 0.0
            )

        m_acc = jnp.broadcast_to(m_col, (ROWS, D))
        l_acc = jnp.broadcast_to(l_col, (ROWS, D))
        send_buf[...] = jnp.concatenate([o_acc, m_acc, l_acc], axis=1)

        barrier_sem = pltpu.get_barrier_semaphore()
        for r in range(1, N_DEV):
            pl.semaphore_signal(
                barrier_sem, inc=1,
                device_id=((my_pos + r) % N_DEV,),
                device_id_type=pl.DeviceIdType.MESH,
            )
        pl.semaphore_wait(barrier_sem, N_DEV - 1)

        rdmas = []
        for r in range(1, N_DEV):
            rdma = pltpu.make_async_remote_copy(
                src_ref=send_buf,
                dst_ref=recv_buf.at[r - 1],
                send_sem=send_sems.at[r - 1],
                recv_sem=recv_sems.at[r - 1],
                device_id=((my_pos + r) % N_DEV,),
                device_id_type=pl.DeviceIdType.MESH,
            )
            rdma.start()
            rdmas.append(rdma)

        for j in (0, 2, 1):
            rdmas[j].wait_recv()
            c = recv_buf[j]
            o_r = c[:, 0:D]
            m_r = c[:, D:2 * D]
            l_r = c[:, 2 * D:3 * D]
            m_new = jnp.maximum(m_acc, m_r)
            a = jnp.exp(m_acc - m_new)
            b = jnp.exp(m_r - m_new)
            o_acc = o_acc * a + o_r * b
            l_acc = l_acc * a + l_r * b
            m_acc = m_new

        for j in range(N_DEV - 1):
            rdmas[j].wait_send()

        out_ref[...] = (o_acc / l_acc).reshape(B, 1, H, D)

    return pl.pallas_call(
        body,
        out_shape=jax.ShapeDtypeStruct((B, 1, H, D), jnp.float32),
        in_specs=[
            pl.BlockSpec(memory_space=pltpu.VMEM),
            pl.BlockSpec(memory_space=pltpu.VMEM),
            pl.BlockSpec(memory_space=pltpu.VMEM),
            pl.BlockSpec(memory_space=pltpu.VMEM),
            pl.BlockSpec(memory_space=pltpu.VMEM),
        ],
        out_specs=pl.BlockSpec(memory_space=pltpu.VMEM),
        scratch_shapes=[
            pltpu.VMEM((ROWS, 3 * D), jnp.float32),
            pltpu.VMEM((3, ROWS, 3 * D), jnp.float32),
            pltpu.SemaphoreType.DMA((3,)),
            pltpu.SemaphoreType.DMA((3,)),
        ],
        compiler_params=pltpu.CompilerParams(collective_id=0),
    )(
        Q.reshape(B, HD),
        K.reshape(T, HD),
        V.reshape(T, HD),
        bt,
        lens.reshape(B, 1),
    )


# device time: 11886 ns/iter; 2.0554x vs baseline; 1.2780x over previous
import jax
import jax.numpy as jnp
from jax import lax
from jax.experimental import pallas as pl
from jax.experimental.pallas import tpu as pltpu

N_DEV = 4
B, H, D, BS = 8, 8, 64, 16
PAGES_PER_SHARD = 64
T = PAGES_PER_SHARD * BS
HD = H * D
ROWS = B * H
NEG_INF = -1e30


def kernel(Q, K, V, bt, lens):
    def body(q_ref, k_ref, v_ref, bt_ref, lens_ref, out_ref,
             send_buf, recv_buf, send_sems, recv_sems):
        my_pos = lax.axis_index("i")

        barrier_sem = pltpu.get_barrier_semaphore()
        for r in range(1, N_DEV):
            pl.semaphore_signal(
                barrier_sem, inc=1,
                device_id=((my_pos + r) % N_DEV,),
                device_id_type=pl.DeviceIdType.MESH,
            )

        q3 = q_ref[...].reshape(B, H, D)
        k2 = k_ref[...]
        v2 = v_ref[...]
        bt_local = bt_ref[...]
        lens_col = lens_ref[...].reshape(B, 1)

        base = my_pos * PAGES_PER_SHARD
        j_iota = lax.broadcasted_iota(jnp.int32, (B, 64, PAGES_PER_SHARD), 1)
        p_iota = lax.broadcasted_iota(jnp.int32, (B, 64, PAGES_PER_SHARD), 2)
        match = (bt_local[:, :, None] == p_iota + base) & (
            j_iota < lens_col[:, :, None]
        )
        w = jnp.sum(match.astype(jnp.float32), axis=1)
        w_tok = jnp.broadcast_to(
            w[:, :, None], (B, PAGES_PER_SHARD, BS)
        ).reshape(B, T)
        w_tok_r = jnp.broadcast_to(
            w_tok[:, None, :], (B, H, T)
        ).reshape(ROWS, T)

        qb4 = jnp.broadcast_to(q3[:, :, None, :], (B, H, H, D))
        h_row = lax.broadcasted_iota(jnp.int32, (B, H, H, D), 1)
        h_lane = lax.broadcasted_iota(jnp.int32, (B, H, H, D), 2)
        q2m = jnp.where(h_row == h_lane, qb4, 0.0).reshape(ROWS, HD)

        s = lax.dot_general(
            q2m, k2, (((1,), (1,)), ((), ())),
            preferred_element_type=jnp.float32,
        ) * (D ** -0.5)
        m_col = jnp.max(s, axis=1, keepdims=True)
        p = jnp.exp(s - m_col) * w_tok_r
        l_col = jnp.sum(p, axis=1, keepdims=True)
        o_full = lax.dot_general(
            p, v2, (((1,), (0,)), ((), ())),
            preferred_element_type=jnp.float32,
        )

        row_h64 = lax.broadcasted_iota(jnp.int32, (ROWS, D), 0) % H
        o_acc = jnp.zeros((ROWS, D), jnp.float32)
        for h in range(H):
            o_acc = o_acc + jnp.where(
                row_h64 == h, o_full[:, h * D:(h + 1) * D], 0.0
            )

        m_acc = m_col
        l_acc = l_col
        send_buf[...] = jnp.concatenate([o_acc, m_col, l_col], axis=1)

        pl.semaphore_wait(barrier_sem, N_DEV - 1)

        rdmas = []
        for r in range(1, N_DEV):
            rdma = pltpu.make_async_remote_copy(
                src_ref=send_buf,
                dst_ref=recv_buf.at[r - 1],
                send_sem=send_sems.at[r - 1],
                recv_sem=recv_sems.at[r - 1],
                device_id=((my_pos + r) % N_DEV,),
                device_id_type=pl.DeviceIdType.MESH,
            )
            rdma.start()
            rdmas.append(rdma)

        for j in (0, 2, 1):
            rdmas[j].wait_recv()
            c = recv_buf[j]
            o_r = c[:, 0:D]
            m_r = c[:, D:D + 1]
            l_r = c[:, D + 1:D + 2]
            m_new = jnp.maximum(m_acc, m_r)
            a = jnp.exp(m_acc - m_new)
            b = jnp.exp(m_r - m_new)
            o_acc = o_acc * a + o_r * b
            l_acc = l_acc * a + l_r * b
            m_acc = m_new

        for j in range(N_DEV - 1):
            rdmas[j].wait_send()

        out_ref[...] = (o_acc / l_acc).reshape(B, 1, H, D)

    return pl.pallas_call(
        body,
        out_shape=jax.ShapeDtypeStruct((B, 1, H, D), jnp.float32),
        in_specs=[
            pl.BlockSpec(memory_space=pltpu.VMEM),
            pl.BlockSpec(memory_space=pltpu.VMEM),
            pl.BlockSpec(memory_space=pltpu.VMEM),
            pl.BlockSpec(memory_space=pltpu.VMEM),
            pl.BlockSpec(memory_space=pltpu.VMEM),
        ],
        out_specs=pl.BlockSpec(memory_space=pltpu.VMEM),
        scratch_shapes=[
            pltpu.VMEM((ROWS, D + 2), jnp.float32),
            pltpu.VMEM((3, ROWS, D + 2), jnp.float32),
            pltpu.SemaphoreType.DMA((3,)),
            pltpu.SemaphoreType.DMA((3,)),
        ],
        compiler_params=pltpu.CompilerParams(collective_id=0),
    )(
        Q,
        K.reshape(T, HD),
        V.reshape(T, HD),
        bt,
        lens,
    )
